# baseline (speedup 1.0000x reference)
import functools

import jax
import jax.numpy as jnp
from jax import lax
from jax.experimental import pallas as pl
from jax.experimental.pallas import tpu as pltpu
from jax.experimental.pallas import tpu_sc as plsc

NC = 2
NS = 16
L = 16
NW = NC * NS

RIN = 16
ROUT = 8
NBUF_IN = 2
NBUF_OUT = 2


def _permute(flat, idx, cp):
    N, COL = flat.shape
    rows_per_w = N // NW
    nchunk = rows_per_w // RIN
    nhalf = RIN // ROUT
    ngrp = COL // L
    assert nchunk % NBUF_IN == 0 and nchunk >= NBUF_IN

    mesh = plsc.VectorSubcoreMesh(
        core_axis_name="c", subcore_axis_name="s", num_cores=NC, num_subcores=NS
    )

    @functools.partial(
        pl.kernel,
        mesh=mesh,
        out_type=jax.ShapeDtypeStruct((N, COL), jnp.float32),
        scratch_types=[
            pltpu.VMEM((COL,), jnp.int32),
            pltpu.VMEM((rows_per_w,), jnp.int32),
            pltpu.VMEM((NBUF_IN, RIN, COL), jnp.float32),
            pltpu.VMEM((NBUF_OUT, ROUT, COL), jnp.float32),
            pltpu.SemaphoreType.DMA((NBUF_IN,)),
            pltpu.SemaphoreType.DMA((NBUF_OUT,)),
        ],
        compiler_params=pltpu.CompilerParams(
            use_tc_tiling_on_sc=False, needs_layout_passes=False
        ),
    )
    def body(flat_hbm, idx_hbm, cp_hbm, out_hbm, cp_v, idx_v, inb, outb, gsem, ssem):
        wid = lax.axis_index("s") * NC + lax.axis_index("c")
        base = wid * rows_per_w
        pltpu.sync_copy(cp_hbm, cp_v)
        pltpu.sync_copy(idx_hbm.at[pl.ds(base, rows_per_w)], idx_v)

        def start_gather(ci, b):
            pltpu.async_copy(
                flat_hbm.at[idx_v.at[pl.ds(ci * RIN, RIN)]], inb.at[b], gsem.at[b]
            )

        def wait_gather(ci, b):
            pltpu.make_async_copy(
                flat_hbm.at[idx_v.at[pl.ds(ci * RIN, RIN)]], inb.at[b], gsem.at[b]
            ).wait()

        def start_scatter(oc, b):
            pltpu.async_copy(
                outb.at[b], out_hbm.at[pl.ds(base + oc * ROUT, ROUT)], ssem.at[b]
            )

        def wait_scatter(oc, b):
            pltpu.make_async_copy(
                outb.at[b], out_hbm.at[pl.ds(base + oc * ROUT, ROUT)], ssem.at[b]
            ).wait()

        rfull = [jnp.full((L,), r, jnp.int32) for r in range(RIN)]

        def compute(bi, half, bo):
            @plsc.parallel_loop(0, ngrp, unroll=2)
            def _grp(j):
                cpj = cp_v[pl.ds(j * L, L)]
                for r in range(ROUT):
                    v = plsc.load_gather(inb.at[bi], [rfull[half * ROUT + r], cpj])
                    outb.at[bo, r].at[pl.ds(j * L, L)].set(v)

        for b in range(NBUF_IN):
            start_gather(b, b)

        def step(t, c):
            for k in range(NBUF_IN):
                ci = NBUF_IN * t + k
                wait_gather(ci, k)
                for half in range(nhalf):
                    oc = nhalf * ci + half
                    bo = (nhalf * k + half) % NBUF_OUT

                    @pl.when(oc >= NBUF_OUT)
                    def _():
                        wait_scatter(oc - NBUF_OUT, bo)

                    compute(k, half, bo)
                    start_scatter(oc, bo)

                @pl.when(ci + NBUF_IN < nchunk)
                def _():
                    start_gather(ci + NBUF_IN, k)

            return c

        lax.fori_loop(0, nchunk // NBUF_IN, step, 0)
        noc = nchunk * nhalf
        for b in range(NBUF_OUT):
            wait_scatter(noc - NBUF_OUT + b, (noc - NBUF_OUT + b) % NBUF_OUT)

    return body(flat, idx, cp)


def _tc_dummy(a):
    def mm(a_ref, o_ref):
        o_ref[...] += jnp.dot(a_ref[...], a_ref[...],
                              preferred_element_type=jnp.float32)

    return pl.pallas_call(
        mm,
        grid=(40,),
        in_specs=[pl.BlockSpec((1024, 1024), lambda i: (0, 0))],
        out_specs=pl.BlockSpec((1024, 1024), lambda i: (0, 0)),
        out_shape=jax.ShapeDtypeStruct((1024, 1024), jnp.float32),
    )(a)


def kernel(tensor, rowperm, colperm):
    B, ROW, COL = tensor.shape
    N = B * ROW
    flat = tensor.reshape(N, COL)
    idx = (
        rowperm.astype(jnp.int32).reshape(1, ROW)
        + (jnp.arange(B, dtype=jnp.int32) * ROW).reshape(B, 1)
    ).reshape(N)
    tc = _tc_dummy(flat[:1024, :1024])
    out = _permute(flat, idx, colperm.astype(jnp.int32))
    out = out.at[0, 0].add(0.0 * tc[0, 0])
    return out.reshape(B, ROW, COL)


# speedup vs baseline: 1.0120x; 1.0120x over previous
import functools

import jax
import jax.numpy as jnp
from jax import lax
from jax.experimental import pallas as pl
from jax.experimental.pallas import tpu as pltpu
from jax.experimental.pallas import tpu_sc as plsc

NC = 2
NS = 16
L = 16
NW = NC * NS

RIN = 16
ROUT = 8
NBUF_IN = 2
NBUF_OUT = 2


def _permute(flat, idx, cp):
    N, COL = flat.shape
    rows_per_w = N // NW
    nchunk = rows_per_w // RIN
    nhalf = RIN // ROUT
    ngrp = COL // L
    assert nchunk % NBUF_IN == 0 and nchunk >= NBUF_IN

    mesh = plsc.VectorSubcoreMesh(
        core_axis_name="c", subcore_axis_name="s", num_cores=NC, num_subcores=NS
    )

    @functools.partial(
        pl.kernel,
        mesh=mesh,
        out_type=jax.ShapeDtypeStruct((N, COL), jnp.float32),
        scratch_types=[
            pltpu.VMEM((COL,), jnp.int32),
            pltpu.VMEM((rows_per_w,), jnp.int32),
            pltpu.VMEM((NBUF_IN, RIN, COL), jnp.float32),
            pltpu.VMEM((NBUF_OUT, ROUT, COL), jnp.float32),
            pltpu.SemaphoreType.DMA((NBUF_IN,)),
            pltpu.SemaphoreType.DMA((NBUF_OUT,)),
        ],
        compiler_params=pltpu.CompilerParams(
            use_tc_tiling_on_sc=False, needs_layout_passes=False
        ),
    )
    def body(flat_hbm, idx_hbm, cp_hbm, out_hbm, cp_v, idx_v, inb, outb, gsem, ssem):
        wid = lax.axis_index("s") * NC + lax.axis_index("c")
        base = wid * rows_per_w
        pltpu.sync_copy(cp_hbm, cp_v)
        pltpu.sync_copy(idx_hbm.at[pl.ds(base, rows_per_w)], idx_v)

        def start_gather(ci, b):
            pltpu.async_copy(
                flat_hbm.at[idx_v.at[pl.ds(ci * RIN, RIN)]], inb.at[b], gsem.at[b]
            )

        def wait_gather(ci, b):
            pltpu.make_async_copy(
                flat_hbm.at[idx_v.at[pl.ds(ci * RIN, RIN)]], inb.at[b], gsem.at[b]
            ).wait()

        def start_scatter(oc, b):
            pltpu.async_copy(
                outb.at[b], out_hbm.at[pl.ds(base + oc * ROUT, ROUT)], ssem.at[b]
            )

        def wait_scatter(oc, b):
            pltpu.make_async_copy(
                outb.at[b], out_hbm.at[pl.ds(base + oc * ROUT, ROUT)], ssem.at[b]
            ).wait()

        rfull = [jnp.full((L,), r, jnp.int32) for r in range(RIN)]

        def compute(bi, half, bo):
            @plsc.parallel_loop(0, ngrp, unroll=2)
            def _grp(j):
                cpj = cp_v[pl.ds(j * L, L)]
                for r in range(ROUT):
                    v = plsc.load_gather(inb.at[bi], [rfull[half * ROUT + r], cpj])
                    outb.at[bo, r].at[pl.ds(j * L, L)].set(v)

        for b in range(NBUF_IN):
            start_gather(b, b)

        def step(t, c):
            for k in range(NBUF_IN):
                ci = NBUF_IN * t + k
                wait_gather(ci, k)
                for half in range(nhalf):
                    oc = nhalf * ci + half
                    bo = (nhalf * k + half) % NBUF_OUT

                    @pl.when(oc >= NBUF_OUT)
                    def _():
                        wait_scatter(oc - NBUF_OUT, bo)

                    compute(k, half, bo)
                    start_scatter(oc, bo)

                @pl.when(ci + NBUF_IN < nchunk)
                def _():
                    start_gather(ci + NBUF_IN, k)

            return c

        lax.fori_loop(0, nchunk // NBUF_IN, step, 0)
        noc = nchunk * nhalf
        for b in range(NBUF_OUT):
            wait_scatter(noc - NBUF_OUT + b, (noc - NBUF_OUT + b) % NBUF_OUT)

    return body(flat, idx, cp)


def kernel(tensor, rowperm, colperm):
    B, ROW, COL = tensor.shape
    N = B * ROW
    flat = tensor.reshape(N, COL)
    idx = (
        rowperm.astype(jnp.int32).reshape(1, ROW)
        + (jnp.arange(B, dtype=jnp.int32) * ROW).reshape(B, 1)
    ).reshape(N)
    out = _permute(flat, idx, colperm.astype(jnp.int32))
    return out.reshape(B, ROW, COL)
